# Initial kernel scaffold; baseline (speedup 1.0000x reference)
import functools

import jax
import jax.numpy as jnp
from jax import lax
from jax.experimental import pallas as pl
from jax.experimental.pallas import tpu as pltpu
from jax.experimental.pallas import tpu_sc as plsc

V = 1000
N_RANGES = 8
ROWS = V // N_RANGES
N_CHUNKS = 4
BLK = 2000
GRPS = BLK // 16


def _sc_partial_counts(xp, zero_slab, n_tokens):
    n_pairs = n_tokens - 1
    chunk = n_tokens // N_CHUNKS
    nblk = chunk // BLK
    mesh = plsc.VectorSubcoreMesh(core_axis_name="c", subcore_axis_name="s")

    @functools.partial(
        pl.kernel,
        out_type=jax.ShapeDtypeStruct((N_CHUNKS, V, V), jnp.float32),
        mesh=mesh,
        scratch_types=[
            pltpu.VMEM((ROWS, V), jnp.float32),
            pltpu.VMEM((BLK + 16,), jnp.int32),
            pltpu.VMEM((BLK + 16,), jnp.int32),
            pltpu.SemaphoreType.DMA,
            pltpu.SemaphoreType.DMA,
        ],
    )
    def sc_kernel(x_hbm, z_hbm, out_hbm, bins, buf0, buf1, sem0, sem1):
        cid = lax.axis_index("c")
        sid = lax.axis_index("s")
        r = sid % N_RANGES
        g = cid * 2 + sid // N_RANGES
        lo = r * ROWS
        tok0 = g * chunk

        pltpu.sync_copy(z_hbm, bins)

        bufs = (buf0, buf1)
        sems = (sem0, sem1)

        def start(kblk, b):
            pltpu.make_async_copy(
                x_hbm.at[pl.ds(tok0 + kblk * BLK, BLK + 16)], bufs[b], sems[b]
            ).start()

        def wait(kblk, b):
            pltpu.make_async_copy(
                x_hbm.at[pl.ds(tok0 + kblk * BLK, BLK + 16)], bufs[b], sems[b]
            ).wait()

        start(0, 0)
        start(1, 1)

        lanes = lax.iota(jnp.int32, 16)
        ones = jnp.ones((16,), jnp.float32)
        lo_v = jnp.full((16,), 1, jnp.int32) * lo
        hi_v = lo_v + ROWS

        def process(buf, kblk):
            pair0 = tok0 + kblk * BLK

            def grp(j, carry):
                off = j * 16
                src = buf[pl.ds(off, 16)]
                dst = plsc.load_gather(buf, [lanes + (off + 1)])
                valid = (pair0 + off + lanes) < n_pairs
                m = (src >= lo_v) & (src < hi_v) & valid
                plsc.addupdate_scatter(bins, [src - lo_v, dst], ones, mask=m)
                return carry

            lax.fori_loop(0, GRPS, grp, 0)

        def outer(i, carry):
            for b in range(2):
                kblk = 2 * i + b
                wait(kblk, b)
                process(bufs[b], kblk)

                @pl.when(kblk + 2 < nblk)
                def _():
                    start(kblk + 2, b)

            return carry

        lax.fori_loop(0, nblk // 2, outer, 0)

        pltpu.sync_copy(bins, out_hbm.at[g, pl.ds(lo, ROWS)])

    return sc_kernel(xp, zero_slab)


def _normalize(partials, w):
    def body(p_ref, w_ref, o_ref):
        c = p_ref[0] + p_ref[1] + p_ref[2] + p_ref[3] + w_ref[...]
        s = jnp.sum(c, axis=1, keepdims=True)
        o_ref[...] = (c + 1.0) / (s + jnp.float32(V))

    return pl.pallas_call(
        body,
        out_shape=jax.ShapeDtypeStruct((V, V), jnp.float32),
    )(partials, w)


@jax.jit
def kernel(X, weight_matrix):
    n = X.shape[0]
    x = X.astype(jnp.int32)
    xp = jnp.concatenate([x, jnp.zeros((16,), jnp.int32)])
    zero_slab = jnp.zeros((ROWS, V), jnp.float32)
    partials = _sc_partial_counts(xp, zero_slab, n)
    return _normalize(partials, weight_matrix)


# speedup vs baseline: 30.3058x; 30.3058x over previous
import functools

import jax
import jax.numpy as jnp
from jax import lax
from jax.experimental import pallas as pl
from jax.experimental.pallas import tpu as pltpu
from jax.experimental.pallas import tpu_sc as plsc

V = 1000
N_RANGES = 8
ROWS = V // N_RANGES
N_CHUNKS = 4
BLK = 2000
GRPS = BLK // 16


def _sc_partial_counts(xp, zero_slab, n_tokens):
    n_pairs = n_tokens - 1
    chunk = n_tokens // N_CHUNKS
    nblk = chunk // BLK
    mesh = plsc.VectorSubcoreMesh(core_axis_name="c", subcore_axis_name="s")

    @functools.partial(
        pl.kernel,
        out_type=jax.ShapeDtypeStruct((N_CHUNKS, V, V), jnp.float32),
        mesh=mesh,
        scratch_types=[
            pltpu.VMEM((ROWS, V), jnp.float32),
            pltpu.VMEM((BLK + 16,), jnp.int32),
            pltpu.VMEM((BLK + 16,), jnp.int32),
            pltpu.SemaphoreType.DMA,
            pltpu.SemaphoreType.DMA,
        ],
        compiler_params=pltpu.CompilerParams(
            use_tc_tiling_on_sc=False, needs_layout_passes=False
        ),
    )
    def sc_kernel(x_hbm, z_hbm, out_hbm, bins, buf0, buf1, sem0, sem1):
        cid = lax.axis_index("c")
        sid = lax.axis_index("s")
        r = sid % N_RANGES
        g = cid * 2 + sid // N_RANGES
        lo = r * ROWS
        tok0 = g * chunk

        pltpu.sync_copy(z_hbm, bins)

        bufs = (buf0, buf1)
        sems = (sem0, sem1)

        def start(kblk, b):
            pltpu.make_async_copy(
                x_hbm.at[pl.ds(tok0 + kblk * BLK, BLK + 16)], bufs[b], sems[b]
            ).start()

        def wait(kblk, b):
            pltpu.make_async_copy(
                x_hbm.at[pl.ds(tok0 + kblk * BLK, BLK + 16)], bufs[b], sems[b]
            ).wait()

        start(0, 0)
        start(1, 1)

        lanes = lax.iota(jnp.int32, 16)
        ones = jnp.ones((16,), jnp.float32)
        lo_v = jnp.full((16,), 1, jnp.int32) * lo
        hi_v = lo_v + ROWS

        def process(buf, kblk):
            pair0 = tok0 + kblk * BLK

            def grp(j, carry):
                off = j * 16
                src = buf[pl.ds(off, 16)]
                dst = plsc.load_gather(buf, [lanes + (off + 1)])
                valid = (pair0 + off + lanes) < n_pairs
                m = (src >= lo_v) & (src < hi_v) & valid
                plsc.addupdate_scatter(bins, [src - lo_v, dst], ones, mask=m)
                return carry

            lax.fori_loop(0, GRPS, grp, 0)

        def outer(i, carry):
            for b in range(2):
                kblk = 2 * i + b
                wait(kblk, b)
                process(bufs[b], kblk)

                @pl.when(kblk + 2 < nblk)
                def _():
                    start(kblk + 2, b)

            return carry

        lax.fori_loop(0, nblk // 2, outer, 0)

        pltpu.sync_copy(bins, out_hbm.at[g, pl.ds(lo, ROWS)])

    return sc_kernel(xp, zero_slab)


def _normalize(partials, w):
    def body(p_ref, w_ref, o_ref):
        c = p_ref[0] + p_ref[1] + p_ref[2] + p_ref[3] + w_ref[...]
        s = jnp.sum(c, axis=1, keepdims=True)
        o_ref[...] = (c + 1.0) / (s + jnp.float32(V))

    return pl.pallas_call(
        body,
        out_shape=jax.ShapeDtypeStruct((V, V), jnp.float32),
    )(partials, w)


@jax.jit
def kernel(X, weight_matrix):
    n = X.shape[0]
    x = X.astype(jnp.int32)
    xp = jnp.concatenate([x, jnp.zeros((16,), jnp.int32)])
    zero_slab = jnp.zeros((ROWS, V), jnp.float32)
    partials = _sc_partial_counts(xp, zero_slab, n)
    return _normalize(partials, weight_matrix)


# speedup vs baseline: 77.8752x; 2.5696x over previous
import functools

import jax
import jax.numpy as jnp
from jax import lax
from jax.experimental import pallas as pl
from jax.experimental.pallas import tpu as pltpu
from jax.experimental.pallas import tpu_sc as plsc

V = 1000
N_RANGES = 8
ROWS = V // N_RANGES
N_CHUNKS = 4
BLK = 4000
JGRPS = BLK // 32
BINS_PAD = 125008


def _sc_partial_counts(x16, n_tokens):
    chunk = n_tokens // N_CHUNKS
    nblk = chunk // BLK
    mesh = plsc.VectorSubcoreMesh(core_axis_name="c", subcore_axis_name="s")

    @functools.partial(
        pl.kernel,
        out_type=jax.ShapeDtypeStruct((N_CHUNKS, N_RANGES, ROWS * V), jnp.float32),
        mesh=mesh,
        scratch_types=[
            pltpu.VMEM((BINS_PAD,), jnp.float32),
            pltpu.VMEM((BLK + 32,), jnp.int16),
            pltpu.VMEM((BLK + 32,), jnp.int16),
            pltpu.VMEM((BLK + 32,), jnp.int16),
            pltpu.SemaphoreType.DMA,
            pltpu.SemaphoreType.DMA,
            pltpu.SemaphoreType.DMA,
        ],
        compiler_params=pltpu.CompilerParams(
            use_tc_tiling_on_sc=False, needs_layout_passes=False
        ),
    )
    def sc_kernel(x_hbm, out_hbm, bins, buf0, buf1, buf2, sem0, sem1, sem2):
        cid = lax.axis_index("c")
        sid = lax.axis_index("s")
        r = sid % N_RANGES
        g = cid * 2 + sid // N_RANGES
        lo = r * ROWS
        tok0 = g * chunk

        bufs = (buf0, buf1, buf2)
        sems = (sem0, sem1, sem2)
        NBUF = 3

        def copy_obj(kblk, b, is_last):
            base = tok0 + kblk * BLK
            if is_last:
                return pltpu.make_async_copy(
                    x_hbm.at[pl.ds(base, BLK)], bufs[b].at[pl.ds(0, BLK)], sems[b]
                )
            return pltpu.make_async_copy(
                x_hbm.at[pl.ds(base, BLK + 32)], bufs[b], sems[b]
            )

        def start(kblk, b):
            last = (g == N_CHUNKS - 1) & (kblk == nblk - 1)

            @pl.when(last)
            def _():
                copy_obj(kblk, b, True).start()

            @pl.when(~last)
            def _():
                copy_obj(kblk, b, False).start()

        def wait(kblk, b):
            last = (g == N_CHUNKS - 1) & (kblk == nblk - 1)

            @pl.when(last)
            def _():
                copy_obj(kblk, b, True).wait()

            @pl.when(~last)
            def _():
                copy_obj(kblk, b, False).wait()

        for b in range(NBUF):
            start(b, b)

        zf = jnp.zeros((16,), jnp.float32)

        def zero_body(i, carry):
            for u in range(13):
                bins[pl.ds((i * 13 + u) * 16, 16)] = zf
            return carry

        lax.fori_loop(0, BINS_PAD // (13 * 16), zero_body, 0)

        lanes = lax.iota(jnp.int32, 16)
        ones = jnp.ones((16,), jnp.float32)
        lo_v = jnp.full((16,), 1, jnp.int32) * lo
        rows_u = jnp.full((16,), ROWS, jnp.uint32)

        def flat_mask(src, dst):
            row = src - lo_v
            flat = row * jnp.int32(V) + dst
            m = plsc.bitcast(row, jnp.uint32) < rows_u
            return flat, m

        def load_j(buf, j):
            e = j * 32
            v0 = buf[pl.ds(e, 32)]
            v1 = buf[pl.ds(e + 2, 32)]
            a0, b0 = plsc.unpack(v0, format=plsc.PackFormat.INTERLEAVED,
                                 preferred_element_type=jnp.int32)
            a1, _ = plsc.unpack(v1, format=plsc.PackFormat.INTERLEAVED,
                                preferred_element_type=jnp.int32)
            return flat_mask(a0, b0), flat_mask(b0, a1)

        UNROLL = 5

        def process(buf):
            def grp(jj, carry):
                pairs = [load_j(buf, jj * UNROLL + u) for u in range(UNROLL)]
                for even, odd in pairs:
                    plsc.addupdate_scatter(bins, [even[0]], ones, mask=even[1])
                    plsc.addupdate_scatter(bins, [odd[0]], ones, mask=odd[1])
                return carry

            lax.fori_loop(0, JGRPS // UNROLL, grp, 0)

        def outer(i, carry):
            for b in range(NBUF):
                kblk = NBUF * i + b

                @pl.when(kblk < nblk)
                def _():
                    wait(kblk, b)
                    process(bufs[b])

                    @pl.when(kblk + NBUF < nblk)
                    def _():
                        start(kblk + NBUF, b)

            return carry

        lax.fori_loop(0, (nblk + NBUF - 1) // NBUF, outer, 0)

        @pl.when(g == N_CHUNKS - 1)
        def _():
            _, (flat, m) = load_j(bufs[(nblk - 1) % NBUF], JGRPS - 1)
            plsc.addupdate_scatter(bins, [flat], -ones, mask=m & (lanes == 15))

        pltpu.sync_copy(bins.at[pl.ds(0, ROWS * V)], out_hbm.at[g, r])

    return sc_kernel(x16)


def _normalize(partials, w):
    def body(p_ref, w_ref, o_ref):
        c = p_ref[0] + p_ref[1] + p_ref[2] + p_ref[3] + w_ref[...]
        s = jnp.sum(c, axis=1, keepdims=True)
        o_ref[...] = (c + 1.0) / (s + jnp.float32(V))

    return pl.pallas_call(
        body,
        out_shape=jax.ShapeDtypeStruct((V, V), jnp.float32),
    )(partials, w)


@jax.jit
def kernel(X, weight_matrix):
    n = X.shape[0]
    x16 = X.astype(jnp.int16)
    partials = _sc_partial_counts(x16, n)
    return _normalize(partials.reshape(N_CHUNKS, V, V), weight_matrix)
